# baseline (speedup 1.0000x reference)
import functools

import jax
import jax.numpy as jnp
from jax import lax
from jax.experimental import pallas as pl
from jax.experimental.pallas import tpu as pltpu
from jax.experimental.pallas import tpu_sc as plsc

EMBED_DIM = 64
BATCH = 4096
HIST = 200

TOTAL = BATCH * HIST
NUM_WORKERS = 32
PER_WORKER = TOTAL // NUM_WORKERS
CHUNK = 800
NUM_CHUNKS = PER_WORKER // CHUNK

_mesh = plsc.VectorSubcoreMesh(core_axis_name="c", subcore_axis_name="s")


@functools.partial(
    pl.kernel,
    mesh=_mesh,
    out_type=jax.ShapeDtypeStruct((TOTAL, EMBED_DIM), jnp.float32),
    scratch_types=[
        pltpu.VMEM((CHUNK,), jnp.int32),
        pltpu.VMEM((CHUNK, EMBED_DIM), jnp.float32),
        pltpu.SemaphoreType.DMA,
    ],
    compiler_params=pltpu.CompilerParams(use_tc_tiling_on_sc=False),
)
def _gather_kernel(idx_hbm, table_hbm, out_hbm, idx_v, rows_v, sem):
    wid = lax.axis_index("s") * 2 + lax.axis_index("c")
    base = wid * PER_WORKER

    def body(i, _):
        off = pl.multiple_of(base + i * CHUNK, 8)
        pltpu.sync_copy(idx_hbm.at[pl.ds(off, CHUNK)], idx_v)
        pltpu.async_copy(table_hbm.at[idx_v], rows_v, sem).wait()
        pltpu.sync_copy(rows_v, out_hbm.at[pl.ds(off, CHUNK)])
        return 0

    lax.fori_loop(0, NUM_CHUNKS, body, 0)


def kernel(input_ids, table):
    flat_ids = input_ids.reshape(TOTAL)
    out = _gather_kernel(flat_ids, table)
    return out.reshape(BATCH, HIST, EMBED_DIM)


# speedup vs baseline: 1.0227x; 1.0227x over previous
import functools

import jax
import jax.numpy as jnp
from jax import lax
from jax.experimental import pallas as pl
from jax.experimental.pallas import tpu as pltpu
from jax.experimental.pallas import tpu_sc as plsc

EMBED_DIM = 64
BATCH = 4096
HIST = 200

TOTAL = BATCH * HIST
NUM_WORKERS = 32
PER_WORKER = TOTAL // NUM_WORKERS
CHUNK = 800
NUM_CHUNKS = PER_WORKER // CHUNK
NBUF = 2

_mesh = plsc.VectorSubcoreMesh(core_axis_name="c", subcore_axis_name="s")


@functools.partial(
    pl.kernel,
    mesh=_mesh,
    out_type=jax.ShapeDtypeStruct((TOTAL, EMBED_DIM), jnp.float32),
    scratch_types=[
        pltpu.VMEM((NUM_CHUNKS, CHUNK), jnp.int32),
        pltpu.VMEM((NBUF, CHUNK, EMBED_DIM), jnp.float32),
        pltpu.SemaphoreType.DMA((NBUF,)),
        pltpu.SemaphoreType.DMA((NBUF,)),
    ],
    compiler_params=pltpu.CompilerParams(use_tc_tiling_on_sc=False),
)
def _gather_kernel(idx_hbm, table_hbm, out_hbm, idx_v, rows_v, gsem, osem):
    wid = lax.axis_index("s") * 2 + lax.axis_index("c")
    base = wid * PER_WORKER

    pltpu.sync_copy(idx_hbm.at[pl.ds(wid * NUM_CHUNKS, NUM_CHUNKS)], idx_v)

    def fire_gather(slot, i):
        pltpu.async_copy(table_hbm.at[idx_v.at[i]], rows_v.at[slot], gsem.at[slot])

    def wait_gather(slot, i):
        pltpu.make_async_copy(
            table_hbm.at[idx_v.at[i]], rows_v.at[slot], gsem.at[slot]
        ).wait()

    def fire_out(slot, i):
        off = pl.multiple_of(base + i * CHUNK, 8)
        pltpu.async_copy(rows_v.at[slot], out_hbm.at[pl.ds(off, CHUNK)], osem.at[slot])

    def wait_out(slot, i):
        off = pl.multiple_of(base + i * CHUNK, 8)
        pltpu.make_async_copy(
            rows_v.at[slot], out_hbm.at[pl.ds(off, CHUNK)], osem.at[slot]
        ).wait()

    for b in range(NBUF):
        fire_gather(b, b)

    def body(g, _):
        for b in range(NBUF):
            i = g * NBUF + b
            wait_gather(b, i)
            fire_out(b, i)
            wait_out(b, i)
            fire_gather(b, i + NBUF)
        return 0

    lax.fori_loop(0, (NUM_CHUNKS - NBUF) // NBUF, body, 0)

    for b in range(NBUF):
        i = NUM_CHUNKS - NBUF + b
        wait_gather(b, i)
        fire_out(b, i)
    for b in range(NBUF):
        i = NUM_CHUNKS - NBUF + b
        wait_out(b, i)


def kernel(input_ids, table):
    flat_ids = input_ids.reshape(NUM_WORKERS * NUM_CHUNKS, CHUNK)
    out = _gather_kernel(flat_ids, table)
    return out.reshape(BATCH, HIST, EMBED_DIM)


# speedup vs baseline: 1.3609x; 1.3307x over previous
import functools

import jax
import jax.numpy as jnp
from jax import lax
from jax.experimental import pallas as pl
from jax.experimental.pallas import tpu as pltpu
from jax.experimental.pallas import tpu_sc as plsc

EMBED_DIM = 64
BATCH = 4096
HIST = 200

TOTAL = BATCH * HIST
NUM_WORKERS = 32
PER_WORKER = TOTAL // NUM_WORKERS
CHUNK = 800
NUM_CHUNKS = PER_WORKER // CHUNK
NBUF = 2

_mesh = plsc.VectorSubcoreMesh(core_axis_name="c", subcore_axis_name="s")


@functools.partial(
    pl.kernel,
    mesh=_mesh,
    out_type=jax.ShapeDtypeStruct((TOTAL, 128), jnp.float32),
    scratch_types=[
        pltpu.VMEM((NUM_CHUNKS, CHUNK), jnp.int32),
        pltpu.VMEM((NBUF, CHUNK, EMBED_DIM), jnp.float32),
        pltpu.SemaphoreType.DMA((NBUF,)),
        pltpu.SemaphoreType.DMA((NBUF,)),
    ],
    compiler_params=pltpu.CompilerParams(use_tc_tiling_on_sc=False),
)
def _gather_kernel(idx_hbm, table_hbm, out_hbm, idx_v, rows_v, gsem, osem):
    wid = lax.axis_index("s") * 2 + lax.axis_index("c")
    base = wid * PER_WORKER

    pltpu.sync_copy(idx_hbm.at[pl.ds(wid * NUM_CHUNKS, NUM_CHUNKS)], idx_v)

    def fire_gather(slot, i):
        pltpu.async_copy(table_hbm.at[idx_v.at[i]], rows_v.at[slot], gsem.at[slot])

    def wait_gather(slot, i):
        pltpu.make_async_copy(
            table_hbm.at[idx_v.at[i]], rows_v.at[slot], gsem.at[slot]
        ).wait()

    def fire_out(slot, i):
        off = pl.multiple_of(base + i * CHUNK, 8)
        pltpu.async_copy(
            rows_v.at[slot],
            out_hbm.at[pl.ds(off, CHUNK), pl.ds(0, EMBED_DIM)],
            osem.at[slot],
        )

    def wait_out(slot, i):
        off = pl.multiple_of(base + i * CHUNK, 8)
        pltpu.make_async_copy(
            rows_v.at[slot],
            out_hbm.at[pl.ds(off, CHUNK), pl.ds(0, EMBED_DIM)],
            osem.at[slot],
        ).wait()

    for b in range(NBUF):
        fire_gather(b, b)

    def body(g, _):
        for b in range(NBUF):
            i = g * NBUF + b
            wait_gather(b, i)
            fire_out(b, i)
            wait_out(b, i)
            fire_gather(b, i + NBUF)
        return 0

    lax.fori_loop(0, (NUM_CHUNKS - NBUF) // NBUF, body, 0)

    for b in range(NBUF):
        i = NUM_CHUNKS - NBUF + b
        wait_gather(b, i)
        fire_out(b, i)
    for b in range(NBUF):
        i = NUM_CHUNKS - NBUF + b
        wait_out(b, i)


def kernel(input_ids, table):
    flat_ids = input_ids.reshape(NUM_WORKERS * NUM_CHUNKS, CHUNK)
    out = _gather_kernel(flat_ids, table)
    return out[:, :EMBED_DIM].reshape(BATCH, HIST, EMBED_DIM)
